# Initial kernel scaffold; baseline (speedup 1.0000x reference)
import jax
import jax.numpy as jnp
from jax.experimental import pallas as pl


def kernel(x_sca, x_pos, edge_index, edge_attr, params):
    raise NotImplementedError("write your pallas kernel here")


# speedup vs baseline: 1.2997x; 1.2997x over previous
import functools

import jax
import jax.numpy as jnp
from jax import lax
from jax.experimental import pallas as pl
from jax.experimental.pallas import tpu as pltpu
from jax.experimental.pallas import tpu_sc as plsc

_F32 = jnp.float32
_I32 = jnp.int32
_NP = 10240
_EP = 327680
_NB = 1280
_EB = 2048
_NW = 32
_CH = 128
_CPW = _EP // (_NW * _CH)
_RPT = _NP // 16
_EPS = 1e-6


def _dot(a, b):
    return lax.dot_general(a, b, (((1,), (0,)), ((), ())),
                           precision=lax.Precision.HIGHEST,
                           preferred_element_type=_F32)


def _leaky(x):
    return jnp.where(x >= 0, x, 0.01 * x)


def _elu(x):
    return jnp.where(x > 0, x, jnp.exp(jnp.minimum(x, 0.0)) - 1.0)


def _vnl(v, wd):
    d = wd * v
    dt = jnp.sum(v * d, axis=-1, keepdims=True)
    dsq = jnp.sum(d * d, axis=-1, keepdims=True)
    adj = v - (dt / (dsq + _EPS)) * d
    return 0.01 * v + 0.99 * jnp.where(dt >= 0, v, adj)


def _gv(s, v, WT, wv, b, g, a1, b1, a2, b2, bg):
    vi = a1 * v + b1
    vn = jnp.sqrt(jnp.sum(vi * vi, axis=-1, keepdims=True))
    so = _dot(s, WT) + vn * wv + b
    gate = jax.nn.sigmoid(_dot(so, g) + bg)
    return so, gate * (a2 * vi + b2)


def _node_pre_body(scal, xs, xp, W1T, w1v, b1, g1, WnT, wnv, bn, gn,
                   WcT, wcv, bc, gc, wn2e, ns_o, small_o, cs_o):
    pos = xp[...]
    s0, v0 = _gv(xs[...], pos, W1T[...], w1v[...], b1[...], g1[...],
                 scal[0], scal[1], scal[2], scal[3], scal[4])
    s1 = _leaky(s0)
    v1 = _vnl(v0, scal[5])
    ns, nv = _gv(s1, v1, WnT[...], wnv[...], bn[...], gn[...],
                 scal[6], scal[7], scal[8], scal[9], scal[10])
    cs, cv = _gv(s1, v1, WcT[...], wcv[...], bc[...], gc[...],
                 scal[11], scal[12], scal[13], scal[14], scal[15])
    n2e = _dot(ns, wn2e[...]) + scal[16]
    z6 = jnp.zeros((pos.shape[0], 6), _F32)
    ns_o[...] = ns
    small_o[...] = jnp.concatenate([pos, nv, n2e, cv, z6], axis=-1)
    cs_o[...] = cs


def _node_pre(scal, xs, xp, W1T, w1v, b1, g1, WnT, wnv, bn, gn,
              WcT, wcv, bc, gc, wn2e):
    full = lambda a: pl.BlockSpec(a.shape, lambda i: (0,) * a.ndim)
    return pl.pallas_call(
        _node_pre_body,
        grid=(_NP // _NB,),
        in_specs=[
            pl.BlockSpec(memory_space=pltpu.SMEM),
            pl.BlockSpec((_NB, 128), lambda i: (i, 0)),
            pl.BlockSpec((_NB, 3), lambda i: (i, 0)),
            full(W1T), full(w1v), full(b1), full(g1),
            full(WnT), full(wnv), full(bn), full(gn),
            full(WcT), full(wcv), full(bc), full(gc), full(wn2e),
        ],
        out_specs=[
            pl.BlockSpec((_NB, 128), lambda i: (i, 0)),
            pl.BlockSpec((_NB, 16), lambda i: (i, 0)),
            pl.BlockSpec((_NB, 128), lambda i: (i, 0)),
        ],
        out_shape=[
            jax.ShapeDtypeStruct((_NP, 128), _F32),
            jax.ShapeDtypeStruct((_NP, 16), _F32),
            jax.ShapeDtypeStruct((_NP, 128), _F32),
        ],
    )(scal, xs, xp, W1T, w1v, b1, g1, WnT, wnv, bn, gn,
      WcT, wcv, bc, gc, wn2e)


def _sc_gather(col, row, ns_tab, smallT):
    mesh = plsc.VectorSubcoreMesh(core_axis_name="c", subcore_axis_name="s")

    @functools.partial(
        pl.kernel,
        out_type=[jax.ShapeDtypeStruct((_EP, 128), _F32),
                  jax.ShapeDtypeStruct((_EP * 16,), _F32)],
        mesh=mesh,
        scratch_types=[
            pltpu.VMEM((_CH,), _I32),
            pltpu.VMEM((_CH,), _I32),
            pltpu.VMEM((8 * _NP,), _F32),
            pltpu.VMEM((_CH, 128), _F32),
            pltpu.VMEM((_CH * 16,), _F32),
            pltpu.SemaphoreType.DMA,
        ],
        compiler_params=pltpu.CompilerParams(needs_layout_passes=False),
    )
    def k(col_h, row_h, tab_h, smt_h, ga_o, se_o,
          idxc, idxr, sbuf, gbuf, obuf, sem):
        wid = lax.axis_index("s") * 2 + lax.axis_index("c")
        base0 = wid * (_CPW * _CH)
        pltpu.sync_copy(smt_h, sbuf)

        def body(j, _):
            base = base0 + j * _CH
            pltpu.sync_copy(col_h.at[pl.ds(base, _CH)], idxc)
            pltpu.sync_copy(row_h.at[pl.ds(base, _CH)], idxr)
            cp = pltpu.async_copy(tab_h.at[idxc], gbuf, sem)
            for k8 in range(_CH // 16):
                ic = idxc[pl.ds(k8 * 16, 16)]
                ir = idxr[pl.ds(k8 * 16, 16)]
                rows = lax.iota(_I32, 16) + (k8 * 16)
                for c in range(7):
                    vc = plsc.load_gather(sbuf, [ic + (c * _NP)])
                    if c < 3:
                        vr = plsc.load_gather(sbuf, [ir + (c * _NP)])
                        val = vr - vc
                    else:
                        val = vc
                    plsc.store_scatter(obuf, [rows * 16 + c], val)
            cp.wait()
            pltpu.sync_copy(gbuf, ga_o.at[pl.ds(base, _CH)])
            pltpu.sync_copy(obuf, se_o.at[pl.ds(base * 16, _CH * 16)])
            return 0

        lax.fori_loop(0, _CPW, body, 0)

    return k(col, row, ns_tab, smallT)


def _edge_body(scal, ga, se, ea, off, WseT, wsev, bse, gse,
               WscaT, bsca, we2n, WsoT, wsov, bso, gso, ms_o, mv_o):
    G = ga[...]
    s = se[...]
    ev = s[:, 0:3]
    nv_c = s[:, 3:6]
    n2e_c = s[:, 6:7]
    ed = jnp.sqrt(jnp.sum(ev * ev, axis=-1, keepdims=True))
    gs = jnp.exp(scal[7] * (ed - off[...]) ** 2)
    evn = ev / (ed + 1e-7)
    evf = scal[6] * evn
    vi_e = scal[0] * evf + scal[1]
    vne = jnp.sqrt(jnp.sum(vi_e * vi_e, axis=-1, keepdims=True))
    es0 = (_dot(gs, WseT[:16, :]) + _dot(ea[...], WseT[16:, :])
           + vne * wsev[...] + bse[...])
    gate_e = jax.nn.sigmoid(_dot(es0, gse[...]) + scal[4])
    evec0 = gate_e * (scal[2] * vi_e + scal[3])
    es = _leaky(es0)
    evec = _vnl(evec0, scal[5])
    A = _dot(es, WscaT[...]) + bsca[...]
    y_sca = G * A
    bb1 = _dot(es, we2n[...]) + scal[8]
    y_vec = bb1 * nv_c + n2e_c * (scal[9] * evec)
    vi_o = scal[10] * y_vec + scal[11]
    vno = jnp.sqrt(jnp.sum(vi_o * vi_o, axis=-1, keepdims=True))
    ms0 = _dot(y_sca, WsoT[...]) + vno * wsov[...] + bso[...]
    gate_o = jax.nn.sigmoid(_dot(ms0, gso[...]) + scal[14])
    mv0 = gate_o * (scal[12] * vi_o + scal[13])
    C = 0.5 * (jnp.cos(ed * scal[15]) + 1.0)
    C = C * (ed <= scal[16]).astype(_F32)
    msf = _elu(ms0 * C)
    mvf = _elu(mv0 * C)
    z = jnp.zeros((mvf.shape[0], 13), _F32)
    ms_o[...] = msf
    mv_o[...] = jnp.concatenate([mvf, z], axis=-1)


def _edge_msg(scal, ga, se, ea, off, WseT, wsev, bse, gse,
              WscaT, bsca, we2n, WsoT, wsov, bso, gso):
    full = lambda a: pl.BlockSpec(a.shape, lambda i: (0,) * a.ndim)
    return pl.pallas_call(
        _edge_body,
        grid=(_EP // _EB,),
        in_specs=[
            pl.BlockSpec(memory_space=pltpu.SMEM),
            pl.BlockSpec((_EB, 128), lambda i: (i, 0)),
            pl.BlockSpec((_EB, 16), lambda i: (i, 0)),
            pl.BlockSpec((_EB, 16), lambda i: (i, 0)),
            full(off), full(WseT), full(wsev), full(bse), full(gse),
            full(WscaT), full(bsca), full(we2n),
            full(WsoT), full(wsov), full(bso), full(gso),
        ],
        out_specs=[
            pl.BlockSpec((_EB, 128), lambda i: (i, 0)),
            pl.BlockSpec((_EB, 16), lambda i: (i, 0)),
        ],
        out_shape=[
            jax.ShapeDtypeStruct((_EP, 128), _F32),
            jax.ShapeDtypeStruct((_EP, 16), _F32),
        ],
    )(scal, ga, se, ea, off, WseT, wsev, bse, gse,
      WscaT, bsca, we2n, WsoT, wsov, bso, gso)


def _sc_scatter_one(row, msgs, z, width, tc_tiling=True):
    mesh = plsc.VectorSubcoreMesh(core_axis_name="c", subcore_axis_name="s")

    @functools.partial(
        pl.kernel,
        out_type=jax.ShapeDtypeStruct((2, _NP, width), _F32),
        mesh=mesh,
        scratch_types=[
            pltpu.VMEM((_CH,), _I32),
            pltpu.VMEM((_CH, width), _F32),
            pltpu.VMEM((32, width), _F32),
            pltpu.VMEM_SHARED((_NP, width), _F32),
        ],
        compiler_params=pltpu.CompilerParams(use_tc_tiling_on_sc=tc_tiling),
    )
    def k(row_h, ms_h, z_h, out_h, idx, mbuf, obuf, acc):
        cid = lax.axis_index("c")
        sid = lax.axis_index("s")
        wid = sid * 2 + cid
        pltpu.sync_copy(z_h, obuf)

        def zb(kk, _):
            pltpu.sync_copy(obuf, acc.at[pl.ds(sid * _RPT + kk * 32, 32)])
            return 0

        lax.fori_loop(0, _RPT // 32, zb, 0)
        plsc.subcore_barrier()
        base0 = wid * (_CPW * _CH)

        def body(j, _):
            base = base0 + j * _CH
            pltpu.sync_copy(row_h.at[pl.ds(base, _CH)], idx)
            pltpu.sync_copy(ms_h.at[pl.ds(base, _CH)], mbuf)
            pltpu.sync_copy(mbuf, acc.at[idx], add=True)
            return 0

        lax.fori_loop(0, _CPW, body, 0)
        plsc.subcore_barrier()

        def cb(kk, _):
            r = sid * _RPT + kk * 32
            pltpu.sync_copy(acc.at[pl.ds(r, 32)], obuf)
            pltpu.sync_copy(obuf, out_h.at[cid, pl.ds(r, 32)])
            return 0

        lax.fori_loop(0, _RPT // 32, cb, 0)

    return k(row, msgs, z)


def _node_out_body(scal, cs, small, part_s, part_v, lnw, lnb, lnw3, lnb3,
                   WtT, wtv, bt, gt, os_o, ov_o):
    ps = part_s[...]
    pv = part_v[...]
    ts = cs[...] + ps[0] + ps[1]
    tv = small[:, 7:10] + (pv[0] + pv[1])[:, 0:3]
    m = jnp.mean(ts, axis=-1, keepdims=True)
    var = jnp.mean((ts - m) ** 2, axis=-1, keepdims=True)
    lns = (ts - m) / jnp.sqrt(var + 1e-5) * lnw[...] + lnb[...]
    mv = jnp.mean(tv, axis=-1, keepdims=True)
    vv = jnp.mean((tv - mv) ** 2, axis=-1, keepdims=True)
    lnv = (tv - mv) / jnp.sqrt(vv + 1e-5) * lnw3[...] + lnb3[...]
    os_ = _leaky(lns)
    ov_ = _vnl(lnv, scal[0])
    fs, fv = _gv(os_, ov_, WtT[...], wtv[...], bt[...], gt[...],
                 scal[1], scal[2], scal[3], scal[4], scal[5])
    os_o[...] = fs
    ov_o[...] = fv


def _node_out(scal, cs, small, part_s, part_v, lnw, lnb, lnw3, lnb3,
              WtT, wtv, bt, gt):
    full = lambda a: pl.BlockSpec(a.shape, lambda i: (0,) * a.ndim)
    return pl.pallas_call(
        _node_out_body,
        grid=(_NP // _NB,),
        in_specs=[
            pl.BlockSpec(memory_space=pltpu.SMEM),
            pl.BlockSpec((_NB, 128), lambda i: (i, 0)),
            pl.BlockSpec((_NB, 16), lambda i: (i, 0)),
            pl.BlockSpec((2, _NB, 128), lambda i: (0, i, 0)),
            pl.BlockSpec((2, _NB, 16), lambda i: (0, i, 0)),
            full(lnw), full(lnb), full(lnw3), full(lnb3),
            full(WtT), full(wtv), full(bt), full(gt),
        ],
        out_specs=[
            pl.BlockSpec((_NB, 128), lambda i: (i, 0)),
            pl.BlockSpec((_NB, 3), lambda i: (i, 0)),
        ],
        out_shape=[
            jax.ShapeDtypeStruct((_NP, 128), _F32),
            jax.ShapeDtypeStruct((_NP, 3), _F32),
        ],
    )(scal, cs, small, part_s, part_v, lnw, lnb, lnw3, lnb3,
      WtT, wtv, bt, gt)


def _gv_views(pp):
    vec = (pp['Ws'][:, 1:].T, pp['Ws'][:, 0][None, :], pp['bs'][None, :],
           pp['Wg'].T)
    sca = [pp['Wv1'][0, 0], pp['bv1'][0], pp['Wv2'][0, 0], pp['bv2'][0],
           pp['bg'][0]]
    return vec, sca


def kernel(x_sca, x_pos, edge_index, edge_attr, params):
    N = x_sca.shape[0]
    E = edge_index.shape[1]
    p = params
    cutoff = 10.0
    edge_dim = 16

    xs = jnp.pad(x_sca, ((0, _NP - N), (0, 0)))
    xp = jnp.pad(x_pos[:, :3], ((0, _NP - N), (0, 0)))
    col = jnp.pad(edge_index[1], (0, _EP - E))
    row_g = jnp.pad(edge_index[0], (0, _EP - E))
    row_s = jnp.pad(edge_index[0], (0, _EP - E), constant_values=N)
    ea = jnp.pad(edge_attr, ((0, _EP - E), (0, 0)))
    zs = jnp.zeros((32, 128), _F32)
    zv = jnp.zeros((32, 16), _F32)

    (v1, s1) = _gv_views(p['per1'])
    (vn, sn) = _gv_views(p['msg_node_gv'])
    (vc, sc) = _gv_views(p['centroid'])
    (ve, se) = _gv_views(p['msg_edge_gvp'])
    (vo, so) = _gv_views(p['msg_out_gv'])
    (vt, st) = _gv_views(p['out_transform'])

    scal_pre = jnp.stack([
        s1[0], s1[1], s1[2], s1[3], s1[4], p['per1']['Wdir'][0, 0],
        sn[0], sn[1], sn[2], sn[3], sn[4],
        sc[0], sc[1], sc[2], sc[3], sc[4],
        p['msg_bn2e'][0],
    ])
    ns_tab, small, cs_ = _node_pre(
        scal_pre, xs, xp,
        v1[0], v1[1], v1[2], v1[3],
        vn[0], vn[1], vn[2], vn[3],
        vc[0], vc[1], vc[2], vc[3],
        p['msg_Wn2e'].T)

    smallT = jnp.pad(small[:, :7].T, ((0, 1), (0, 0))).reshape(-1)

    ga, se_flat = _sc_gather(col, row_g, ns_tab, smallT)
    se_feat = se_flat.reshape(_EP, 16)

    off = jnp.linspace(0.0, cutoff, edge_dim)[None, :].astype(_F32)
    coeff = -0.5 / (cutoff / (edge_dim - 1)) ** 2
    scal_edge = jnp.stack([
        se[0], se[1], se[2], se[3], se[4],
        p['msg_edge_gvp']['Wdir'][0, 0],
        p['vec_exp_W'][0, 0],
        jnp.float32(coeff),
        p['msg_be2n'][0],
        p['msg_Wevn'][0, 0],
        so[0], so[1], so[2], so[3], so[4],
        jnp.float32(jnp.pi / cutoff),
        jnp.float32(cutoff),
    ])
    ms, mv = _edge_msg(
        scal_edge, ga, se_feat, ea, off,
        ve[0], ve[1], ve[2], ve[3],
        p['msg_Wsca'].T, p['msg_bsca'][None, :], p['msg_We2n'].T,
        vo[0], vo[1], vo[2], vo[3])

    part_s = _sc_scatter_one(row_s, ms, zs, 128)
    part_v = _sc_scatter_one(row_s, mv, zv, 16, tc_tiling=False)

    scal_out = jnp.stack([
        p['act_vec_Wdir'][0, 0],
        st[0], st[1], st[2], st[3], st[4],
    ])
    os_, ov_ = _node_out(
        scal_out, cs_, small, part_s, part_v,
        p['ln_sca_w'][None, :], p['ln_sca_b'][None, :],
        p['ln_vec_w'], p['ln_vec_b'],
        vt[0], vt[1], vt[2], vt[3])

    return os_[:N], ov_[:N].reshape(N, 1, 3)


# speedup vs baseline: 2.0084x; 1.5453x over previous
import functools

import jax
import jax.numpy as jnp
from jax import lax
from jax.experimental import pallas as pl
from jax.experimental.pallas import tpu as pltpu
from jax.experimental.pallas import tpu_sc as plsc

_F32 = jnp.float32
_I32 = jnp.int32
_NP = 10240
_EP = 327680
_NB = 1280
_EB = 2048
_NW = 32
_CH = 128
_CPW = _EP // (_NW * _CH)
_RPT = _NP // 16
_EPS = 1e-6


def _dot(a, b, precision=lax.Precision.HIGHEST):
    return lax.dot_general(a, b, (((1,), (0,)), ((), ())),
                           precision=precision,
                           preferred_element_type=_F32)


def _dotd(a, b):
    return _dot(a, b, precision=lax.Precision.DEFAULT)


def _leaky(x):
    return jnp.where(x >= 0, x, 0.01 * x)


def _elu(x):
    return jnp.where(x > 0, x, jnp.exp(jnp.minimum(x, 0.0)) - 1.0)


def _vnl(v, wd):
    d = wd * v
    dt = jnp.sum(v * d, axis=-1, keepdims=True)
    dsq = jnp.sum(d * d, axis=-1, keepdims=True)
    adj = v - (dt / (dsq + _EPS)) * d
    return 0.01 * v + 0.99 * jnp.where(dt >= 0, v, adj)


def _gv(s, v, WT, wv, b, g, a1, b1, a2, b2, bg):
    vi = a1 * v + b1
    vn = jnp.sqrt(jnp.sum(vi * vi, axis=-1, keepdims=True))
    so = _dot(s, WT) + vn * wv + b
    gate = jax.nn.sigmoid(_dot(so, g) + bg)
    return so, gate * (a2 * vi + b2)


def _node_pre_body(scal, xs, xp, W1T, w1v, b1, g1, WnT, wnv, bn, gn,
                   WcT, wcv, bc, gc, wn2e, ns_o, small_o, cs_o):
    pos = xp[...]
    s0, v0 = _gv(xs[...], pos, W1T[...], w1v[...], b1[...], g1[...],
                 scal[0], scal[1], scal[2], scal[3], scal[4])
    s1 = _leaky(s0)
    v1 = _vnl(v0, scal[5])
    ns, nv = _gv(s1, v1, WnT[...], wnv[...], bn[...], gn[...],
                 scal[6], scal[7], scal[8], scal[9], scal[10])
    cs, cv = _gv(s1, v1, WcT[...], wcv[...], bc[...], gc[...],
                 scal[11], scal[12], scal[13], scal[14], scal[15])
    n2e = _dot(ns, wn2e[...]) + scal[16]
    z6 = jnp.zeros((pos.shape[0], 6), _F32)
    ns_o[...] = ns
    small_o[...] = jnp.concatenate([pos, nv, n2e, cv, z6], axis=-1)
    cs_o[...] = cs


def _node_pre(scal, xs, xp, W1T, w1v, b1, g1, WnT, wnv, bn, gn,
              WcT, wcv, bc, gc, wn2e):
    full = lambda a: pl.BlockSpec(a.shape, lambda i: (0,) * a.ndim)
    return pl.pallas_call(
        _node_pre_body,
        grid=(_NP // _NB,),
        in_specs=[
            pl.BlockSpec(memory_space=pltpu.SMEM),
            pl.BlockSpec((_NB, 128), lambda i: (i, 0)),
            pl.BlockSpec((_NB, 3), lambda i: (i, 0)),
            full(W1T), full(w1v), full(b1), full(g1),
            full(WnT), full(wnv), full(bn), full(gn),
            full(WcT), full(wcv), full(bc), full(gc), full(wn2e),
        ],
        out_specs=[
            pl.BlockSpec((_NB, 128), lambda i: (i, 0)),
            pl.BlockSpec((_NB, 16), lambda i: (i, 0)),
            pl.BlockSpec((_NB, 128), lambda i: (i, 0)),
        ],
        out_shape=[
            jax.ShapeDtypeStruct((_NP, 128), _F32),
            jax.ShapeDtypeStruct((_NP, 16), _F32),
            jax.ShapeDtypeStruct((_NP, 128), _F32),
        ],
    )(scal, xs, xp, W1T, w1v, b1, g1, WnT, wnv, bn, gn,
      WcT, wcv, bc, gc, wn2e)


def _sc_gather(col, row, ns_tab, smallT):
    mesh = plsc.VectorSubcoreMesh(core_axis_name="c", subcore_axis_name="s")

    @functools.partial(
        pl.kernel,
        out_type=[jax.ShapeDtypeStruct((_EP, 128), _F32),
                  jax.ShapeDtypeStruct((_EP * 16,), _F32)],
        mesh=mesh,
        scratch_types=[
            pltpu.VMEM((_CH,), _I32),
            pltpu.VMEM((_CH,), _I32),
            pltpu.VMEM((8 * _NP,), _F32),
            pltpu.VMEM((_CH, 128), _F32),
            pltpu.VMEM((_CH * 16,), _F32),
            pltpu.SemaphoreType.DMA,
        ],
        compiler_params=pltpu.CompilerParams(needs_layout_passes=False),
    )
    def k(col_h, row_h, tab_h, smt_h, ga_o, se_o,
          idxc, idxr, sbuf, gbuf, obuf, sem):
        wid = lax.axis_index("s") * 2 + lax.axis_index("c")
        base0 = wid * (_CPW * _CH)
        pltpu.sync_copy(smt_h, sbuf)

        def body(j, _):
            base = base0 + j * _CH
            pltpu.sync_copy(col_h.at[pl.ds(base, _CH)], idxc)
            pltpu.sync_copy(row_h.at[pl.ds(base, _CH)], idxr)
            cp = pltpu.async_copy(tab_h.at[idxc], gbuf, sem)
            for k8 in range(_CH // 16):
                ic = idxc[pl.ds(k8 * 16, 16)]
                ir = idxr[pl.ds(k8 * 16, 16)]
                rows = lax.iota(_I32, 16) + (k8 * 16)
                for c in range(7):
                    vc = plsc.load_gather(sbuf, [ic + (c * _NP)])
                    if c < 3:
                        vr = plsc.load_gather(sbuf, [ir + (c * _NP)])
                        val = vr - vc
                    else:
                        val = vc
                    plsc.store_scatter(obuf, [rows * 16 + c], val)
            cp.wait()
            pltpu.sync_copy(gbuf, ga_o.at[pl.ds(base, _CH)])
            pltpu.sync_copy(obuf, se_o.at[pl.ds(base * 16, _CH * 16)])
            return 0

        lax.fori_loop(0, _CPW, body, 0)

    return k(col, row, ns_tab, smallT)


def _edge_body(scal, ga, se, ea, off, WseT, wsev, bse, gse,
               WscaT, bsca, we2n, WsoT, wsov, bso, gso, ms_o, mv_o):
    G = ga[...]
    s = se[...]
    ev = s[:, 0:3]
    nv_c = s[:, 3:6]
    n2e_c = s[:, 6:7]
    ed = jnp.sqrt(jnp.sum(ev * ev, axis=-1, keepdims=True))
    gs = jnp.exp(scal[7] * (ed - off[...]) ** 2)
    evn = ev / (ed + 1e-7)
    evf = scal[6] * evn
    vi_e = scal[0] * evf + scal[1]
    vne = jnp.sqrt(jnp.sum(vi_e * vi_e, axis=-1, keepdims=True))
    es0 = (_dotd(gs, WseT[:16, :]) + _dotd(ea[...], WseT[16:, :])
           + vne * wsev[...] + bse[...])
    gate_e = jax.nn.sigmoid(_dotd(es0, gse[...]) + scal[4])
    evec0 = gate_e * (scal[2] * vi_e + scal[3])
    es = _leaky(es0)
    evec = _vnl(evec0, scal[5])
    A = _dotd(es, WscaT[...]) + bsca[...]
    y_sca = G * A
    bb1 = _dotd(es, we2n[...]) + scal[8]
    y_vec = bb1 * nv_c + n2e_c * (scal[9] * evec)
    vi_o = scal[10] * y_vec + scal[11]
    vno = jnp.sqrt(jnp.sum(vi_o * vi_o, axis=-1, keepdims=True))
    ms0 = _dotd(y_sca, WsoT[...]) + vno * wsov[...] + bso[...]
    gate_o = jax.nn.sigmoid(_dotd(ms0, gso[...]) + scal[14])
    mv0 = gate_o * (scal[12] * vi_o + scal[13])
    C = 0.5 * (jnp.cos(ed * scal[15]) + 1.0)
    C = C * (ed <= scal[16]).astype(_F32)
    msf = _elu(ms0 * C)
    mvf = _elu(mv0 * C)
    z = jnp.zeros((mvf.shape[0], 13), _F32)
    ms_o[...] = msf
    mv_o[...] = jnp.concatenate([mvf, z], axis=-1)


def _edge_msg(scal, ga, se, ea, off, WseT, wsev, bse, gse,
              WscaT, bsca, we2n, WsoT, wsov, bso, gso):
    full = lambda a: pl.BlockSpec(a.shape, lambda i: (0,) * a.ndim)
    return pl.pallas_call(
        _edge_body,
        grid=(_EP // _EB,),
        in_specs=[
            pl.BlockSpec(memory_space=pltpu.SMEM),
            pl.BlockSpec((_EB, 128), lambda i: (i, 0)),
            pl.BlockSpec((_EB, 16), lambda i: (i, 0)),
            pl.BlockSpec((_EB, 16), lambda i: (i, 0)),
            full(off), full(WseT), full(wsev), full(bse), full(gse),
            full(WscaT), full(bsca), full(we2n),
            full(WsoT), full(wsov), full(bso), full(gso),
        ],
        out_specs=[
            pl.BlockSpec((_EB, 128), lambda i: (i, 0)),
            pl.BlockSpec((_EB, 16), lambda i: (i, 0)),
        ],
        out_shape=[
            jax.ShapeDtypeStruct((_EP, 128), _F32),
            jax.ShapeDtypeStruct((_EP, 16), _F32),
        ],
    )(scal, ga, se, ea, off, WseT, wsev, bse, gse,
      WscaT, bsca, we2n, WsoT, wsov, bso, gso)


def _sc_scatter_one(row, msgs, z, width, tc_tiling=True):
    mesh = plsc.VectorSubcoreMesh(core_axis_name="c", subcore_axis_name="s")

    @functools.partial(
        pl.kernel,
        out_type=jax.ShapeDtypeStruct((2, _NP, width), _F32),
        mesh=mesh,
        scratch_types=[
            pltpu.VMEM((_CH,), _I32),
            pltpu.VMEM((_CH, width), _F32),
            pltpu.VMEM((32, width), _F32),
            pltpu.VMEM_SHARED((_NP, width), _F32),
        ],
        compiler_params=pltpu.CompilerParams(use_tc_tiling_on_sc=tc_tiling),
    )
    def k(row_h, ms_h, z_h, out_h, idx, mbuf, obuf, acc):
        cid = lax.axis_index("c")
        sid = lax.axis_index("s")
        wid = sid * 2 + cid
        pltpu.sync_copy(z_h, obuf)

        def zb(kk, _):
            pltpu.sync_copy(obuf, acc.at[pl.ds(sid * _RPT + kk * 32, 32)])
            return 0

        lax.fori_loop(0, _RPT // 32, zb, 0)
        plsc.subcore_barrier()
        base0 = wid * (_CPW * _CH)

        def body(j, _):
            base = base0 + j * _CH
            pltpu.sync_copy(row_h.at[pl.ds(base, _CH)], idx)
            pltpu.sync_copy(ms_h.at[pl.ds(base, _CH)], mbuf)
            pltpu.sync_copy(mbuf, acc.at[idx], add=True)
            return 0

        lax.fori_loop(0, _CPW, body, 0)
        plsc.subcore_barrier()

        def cb(kk, _):
            r = sid * _RPT + kk * 32
            pltpu.sync_copy(acc.at[pl.ds(r, 32)], obuf)
            pltpu.sync_copy(obuf, out_h.at[cid, pl.ds(r, 32)])
            return 0

        lax.fori_loop(0, _RPT // 32, cb, 0)

    return k(row, msgs, z)


def _node_out_body(scal, cs, small, part_s, part_v, lnw, lnb, lnw3, lnb3,
                   WtT, wtv, bt, gt, os_o, ov_o):
    ps = part_s[...]
    pv = part_v[...]
    ts = cs[...] + ps[0] + ps[1]
    tv = small[:, 7:10] + (pv[0] + pv[1])[:, 0:3]
    m = jnp.mean(ts, axis=-1, keepdims=True)
    var = jnp.mean((ts - m) ** 2, axis=-1, keepdims=True)
    lns = (ts - m) / jnp.sqrt(var + 1e-5) * lnw[...] + lnb[...]
    mv = jnp.mean(tv, axis=-1, keepdims=True)
    vv = jnp.mean((tv - mv) ** 2, axis=-1, keepdims=True)
    lnv = (tv - mv) / jnp.sqrt(vv + 1e-5) * lnw3[...] + lnb3[...]
    os_ = _leaky(lns)
    ov_ = _vnl(lnv, scal[0])
    fs, fv = _gv(os_, ov_, WtT[...], wtv[...], bt[...], gt[...],
                 scal[1], scal[2], scal[3], scal[4], scal[5])
    os_o[...] = fs
    ov_o[...] = fv


def _node_out(scal, cs, small, part_s, part_v, lnw, lnb, lnw3, lnb3,
              WtT, wtv, bt, gt):
    full = lambda a: pl.BlockSpec(a.shape, lambda i: (0,) * a.ndim)
    return pl.pallas_call(
        _node_out_body,
        grid=(_NP // _NB,),
        in_specs=[
            pl.BlockSpec(memory_space=pltpu.SMEM),
            pl.BlockSpec((_NB, 128), lambda i: (i, 0)),
            pl.BlockSpec((_NB, 16), lambda i: (i, 0)),
            pl.BlockSpec((2, _NB, 128), lambda i: (0, i, 0)),
            pl.BlockSpec((2, _NB, 16), lambda i: (0, i, 0)),
            full(lnw), full(lnb), full(lnw3), full(lnb3),
            full(WtT), full(wtv), full(bt), full(gt),
        ],
        out_specs=[
            pl.BlockSpec((_NB, 128), lambda i: (i, 0)),
            pl.BlockSpec((_NB, 3), lambda i: (i, 0)),
        ],
        out_shape=[
            jax.ShapeDtypeStruct((_NP, 128), _F32),
            jax.ShapeDtypeStruct((_NP, 3), _F32),
        ],
    )(scal, cs, small, part_s, part_v, lnw, lnb, lnw3, lnb3,
      WtT, wtv, bt, gt)


def _gv_views(pp):
    vec = (pp['Ws'][:, 1:].T, pp['Ws'][:, 0][None, :], pp['bs'][None, :],
           pp['Wg'].T)
    sca = [pp['Wv1'][0, 0], pp['bv1'][0], pp['Wv2'][0, 0], pp['bv2'][0],
           pp['bg'][0]]
    return vec, sca


def kernel(x_sca, x_pos, edge_index, edge_attr, params):
    N = x_sca.shape[0]
    E = edge_index.shape[1]
    p = params
    cutoff = 10.0
    edge_dim = 16

    xs = jnp.pad(x_sca, ((0, _NP - N), (0, 0)))
    xp = jnp.pad(x_pos[:, :3], ((0, _NP - N), (0, 0)))
    col = jnp.pad(edge_index[1], (0, _EP - E))
    row_g = jnp.pad(edge_index[0], (0, _EP - E))
    row_s = jnp.pad(edge_index[0], (0, _EP - E), constant_values=N)
    ea = jnp.pad(edge_attr, ((0, _EP - E), (0, 0)))
    zs = jnp.zeros((32, 128), _F32)
    zv = jnp.zeros((32, 16), _F32)

    (v1, s1) = _gv_views(p['per1'])
    (vn, sn) = _gv_views(p['msg_node_gv'])
    (vc, sc) = _gv_views(p['centroid'])
    (ve, se) = _gv_views(p['msg_edge_gvp'])
    (vo, so) = _gv_views(p['msg_out_gv'])
    (vt, st) = _gv_views(p['out_transform'])

    scal_pre = jnp.stack([
        s1[0], s1[1], s1[2], s1[3], s1[4], p['per1']['Wdir'][0, 0],
        sn[0], sn[1], sn[2], sn[3], sn[4],
        sc[0], sc[1], sc[2], sc[3], sc[4],
        p['msg_bn2e'][0],
    ])
    ns_tab, small, cs_ = _node_pre(
        scal_pre, xs, xp,
        v1[0], v1[1], v1[2], v1[3],
        vn[0], vn[1], vn[2], vn[3],
        vc[0], vc[1], vc[2], vc[3],
        p['msg_Wn2e'].T)

    smallT = jnp.pad(small[:, :7].T, ((0, 1), (0, 0))).reshape(-1)

    ga, se_flat = _sc_gather(col, row_g, ns_tab, smallT)
    se_feat = se_flat.reshape(_EP, 16)

    off = jnp.linspace(0.0, cutoff, edge_dim)[None, :].astype(_F32)
    coeff = -0.5 / (cutoff / (edge_dim - 1)) ** 2
    scal_edge = jnp.stack([
        se[0], se[1], se[2], se[3], se[4],
        p['msg_edge_gvp']['Wdir'][0, 0],
        p['vec_exp_W'][0, 0],
        jnp.float32(coeff),
        p['msg_be2n'][0],
        p['msg_Wevn'][0, 0],
        so[0], so[1], so[2], so[3], so[4],
        jnp.float32(jnp.pi / cutoff),
        jnp.float32(cutoff),
    ])
    ms, mv = _edge_msg(
        scal_edge, ga, se_feat, ea, off,
        ve[0], ve[1], ve[2], ve[3],
        p['msg_Wsca'].T, p['msg_bsca'][None, :], p['msg_We2n'].T,
        vo[0], vo[1], vo[2], vo[3])

    part_s = _sc_scatter_one(row_s, ms, zs, 128)
    part_v = _sc_scatter_one(row_s, mv, zv, 16, tc_tiling=False)

    scal_out = jnp.stack([
        p['act_vec_Wdir'][0, 0],
        st[0], st[1], st[2], st[3], st[4],
    ])
    os_, ov_ = _node_out(
        scal_out, cs_, small, part_s, part_v,
        p['ln_sca_w'][None, :], p['ln_sca_b'][None, :],
        p['ln_vec_w'], p['ln_vec_b'],
        vt[0], vt[1], vt[2], vt[3])

    return os_[:N], ov_[:N].reshape(N, 1, 3)
